# Initial kernel scaffold; baseline (speedup 1.0000x reference)
import jax
import jax.numpy as jnp
from jax.experimental import pallas as pl


def kernel(position_ids, cos_cache, sin_cache):
    raise NotImplementedError("write your pallas kernel here")


# speedup vs baseline: 3.0182x; 3.0182x over previous
import jax
import jax.numpy as jnp
from jax import lax
from jax.experimental import pallas as pl
from jax.experimental.pallas import tpu as pltpu
from jax.experimental.pallas import tpu_sc as plsc

BATCH = 4
SEQ = 8192
D = 64
B = BATCH * SEQ
NC = 2
NS = 16
NW = NC * NS
B_PER_W = B // NW
CHUNK = 128
NCHUNK = B_PER_W // CHUNK
IDX_ROWS = B // CHUNK


def _rope_gather(idx_hbm, cos_hbm, sin_hbm, cos_out, sin_out,
                 idx_v, cos_v, sin_v, sem):
    c = lax.axis_index("c")
    s = lax.axis_index("s")
    wid = s * NC + c
    row0 = wid * NCHUNK
    pltpu.sync_copy(idx_hbm.at[pl.ds(row0, NCHUNK)], idx_v)
    for j in range(NCHUNK):
        ccopy = pltpu.async_copy(cos_hbm.at[idx_v.at[j]], cos_v, sem)
        scopy = pltpu.async_copy(sin_hbm.at[idx_v.at[j]], sin_v, sem)
        ccopy.wait()
        scopy.wait()
        base = (row0 + j) * CHUNK
        pltpu.sync_copy(cos_v, cos_out.at[pl.ds(base, CHUNK)])
        pltpu.sync_copy(sin_v, sin_out.at[pl.ds(base, CHUNK)])


def kernel(position_ids, cos_cache, sin_cache):
    idx2d = position_ids.reshape(IDX_ROWS, CHUNK)
    mesh = plsc.VectorSubcoreMesh(core_axis_name="c", subcore_axis_name="s")
    run = pl.kernel(
        _rope_gather,
        out_type=(
            jax.ShapeDtypeStruct((B, D), jnp.float32),
            jax.ShapeDtypeStruct((B, D), jnp.float32),
        ),
        mesh=mesh,
        scratch_types=[
            pltpu.VMEM((NCHUNK, CHUNK), jnp.int32),
            pltpu.VMEM((CHUNK, D), jnp.float32),
            pltpu.VMEM((CHUNK, D), jnp.float32),
            pltpu.SemaphoreType.DMA,
        ],
        compiler_params=pltpu.CompilerParams(use_tc_tiling_on_sc=False),
    )
    cos, sin = run(idx2d, cos_cache, sin_cache)
    return cos.reshape(BATCH, SEQ, D), sin.reshape(BATCH, SEQ, D)


# speedup vs baseline: 3.1743x; 1.0517x over previous
import jax
import jax.numpy as jnp
from jax import lax
from jax.experimental import pallas as pl
from jax.experimental.pallas import tpu as pltpu
from jax.experimental.pallas import tpu_sc as plsc

BATCH = 4
SEQ = 8192
D = 64
B = BATCH * SEQ
NC = 2
NS = 16
NW = NC * NS
B_PER_W = B // NW
CHUNK = 128
NCHUNK = B_PER_W // CHUNK
IDX_ROWS = B // CHUNK


NBUF = 2


def _rope_gather(idx_hbm, cos_hbm, sin_hbm, cos_out, sin_out,
                 idx_v, cos_v, sin_v, gsem0, gsem1, wsem0, wsem1):
    gsem = (gsem0, gsem1)
    wsem = (wsem0, wsem1)
    c = lax.axis_index("c")
    s = lax.axis_index("s")
    wid = s * NC + c
    row0 = wid * NCHUNK
    pltpu.sync_copy(idx_hbm.at[pl.ds(row0, NCHUNK)], idx_v)
    gathers = [None] * NCHUNK
    writes = [None] * NCHUNK
    for j in range(NCHUNK):
        b = j % NBUF
        if j >= NBUF:
            writes[j - NBUF][0].wait()
            writes[j - NBUF][1].wait()
        gathers[j] = (
            pltpu.async_copy(cos_hbm.at[idx_v.at[j]], cos_v.at[b], gsem[b]),
            pltpu.async_copy(sin_hbm.at[idx_v.at[j]], sin_v.at[b], gsem[b]),
        )
        if j >= 1:
            pb = (j - 1) % NBUF
            gathers[j - 1][0].wait()
            gathers[j - 1][1].wait()
            base = (row0 + j - 1) * CHUNK
            writes[j - 1] = (
                pltpu.async_copy(cos_v.at[pb], cos_out.at[pl.ds(base, CHUNK)],
                                 wsem[pb]),
                pltpu.async_copy(sin_v.at[pb], sin_out.at[pl.ds(base, CHUNK)],
                                 wsem[pb]),
            )
    j = NCHUNK - 1
    b = j % NBUF
    gathers[j][0].wait()
    gathers[j][1].wait()
    base = (row0 + j) * CHUNK
    writes[j] = (
        pltpu.async_copy(cos_v.at[b], cos_out.at[pl.ds(base, CHUNK)], wsem[b]),
        pltpu.async_copy(sin_v.at[b], sin_out.at[pl.ds(base, CHUNK)], wsem[b]),
    )
    for j in (NCHUNK - 2, NCHUNK - 1):
        writes[j][0].wait()
        writes[j][1].wait()


def kernel(position_ids, cos_cache, sin_cache):
    idx2d = position_ids.reshape(IDX_ROWS, CHUNK)
    mesh = plsc.VectorSubcoreMesh(core_axis_name="c", subcore_axis_name="s")
    run = pl.kernel(
        _rope_gather,
        out_type=(
            jax.ShapeDtypeStruct((B, D), jnp.float32),
            jax.ShapeDtypeStruct((B, D), jnp.float32),
        ),
        mesh=mesh,
        scratch_types=[
            pltpu.VMEM((NCHUNK, CHUNK), jnp.int32),
            pltpu.VMEM((NBUF, CHUNK, D), jnp.float32),
            pltpu.VMEM((NBUF, CHUNK, D), jnp.float32),
            pltpu.SemaphoreType.DMA,
            pltpu.SemaphoreType.DMA,
            pltpu.SemaphoreType.DMA,
            pltpu.SemaphoreType.DMA,
        ],
        compiler_params=pltpu.CompilerParams(use_tc_tiling_on_sc=False),
    )
    cos, sin = run(idx2d, cos_cache, sin_cache)
    return cos.reshape(BATCH, SEQ, D), sin.reshape(BATCH, SEQ, D)
